# baseline (speedup 1.0000x reference)
import functools

import numpy as np
import jax
import jax.numpy as jnp
from jax import lax
from jax.experimental import pallas as pl
from jax.experimental.pallas import tpu as pltpu
from jax.experimental.pallas import tpu_sc as plsc

B = 64
K = 100
KP = 128
H = 512.0
W = 512.0
NC = 2
NS = 16
NW = NC * NS
SPW = B // NW
NCHUNK = KP // 16
ROW = 5 * KP


def _rotl(x, d):
    return ((x << np.uint32(d)) | (x >> np.uint32(32 - d))).astype(np.uint32)


def _threefry2x32(k0, k1, x0, x1):
    x0 = x0.astype(np.uint32).copy()
    x1 = x1.astype(np.uint32).copy()
    ks = [np.uint32(k0), np.uint32(k1),
          np.uint32(k0) ^ np.uint32(k1) ^ np.uint32(0x1BD11BDA)]
    rot = [[13, 15, 26, 6], [17, 29, 16, 24]]
    x0 = (x0 + ks[0]).astype(np.uint32)
    x1 = (x1 + ks[1]).astype(np.uint32)
    for i in range(5):
        for r in rot[i % 2]:
            x0 = (x0 + x1).astype(np.uint32)
            x1 = _rotl(x1, r)
            x1 = (x1 ^ x0).astype(np.uint32)
        x0 = (x0 + ks[(i + 1) % 3]).astype(np.uint32)
        x1 = (x1 + ks[(i + 2) % 3] + np.uint32(i + 1)).astype(np.uint32)
    return x0, x1


def _splitn(k, n):
    c = np.arange(n, dtype=np.uint32)
    a, b = _threefry2x32(k[0], k[1], np.zeros(n, np.uint32), c)
    return np.stack([a, b], -1)


def _bits(k, n):
    c = np.arange(n, dtype=np.uint32)
    a, b = _threefry2x32(k[0], k[1], np.zeros(n, np.uint32), c)
    return (a ^ b).astype(np.uint32)


def _uniform01(bits):
    m = (bits >> np.uint32(9)) | np.uint32(0x3F800000)
    return np.maximum(m.view(np.float32) - np.float32(1.0), np.float32(0.0))


def _rng_constants():
    base = np.array([0, 42], dtype=np.uint32)
    ks = _splitn(base, 5)
    k1, k2 = _splitn(ks[0], 2)
    hb, lb = _bits(k1, B), _bits(k2, B)
    span = np.uint32(B - 1)
    mult = np.uint32((2**32) % (B - 1))
    r = (((hb % span) * mult + (lb % span)) % span).astype(np.int32)
    src = (r + (r >= np.arange(B, dtype=np.int32)).astype(np.int32)).astype(np.int32)
    scale = _uniform01(_bits(ks[1], B)) * np.float32(1.5) + np.float32(0.5)
    u_ty = _uniform01(_bits(ks[2], B))
    u_tx = _uniform01(_bits(ks[3], B))
    hflip = _uniform01(_bits(ks[4], B)) < np.float32(0.5)
    return src, scale, u_ty, u_tx, hflip


_SRC, _SCALE, _UTY, _UTX, _HFLIP = _rng_constants()

_IDX_PAD = np.zeros((NW, 8), dtype=np.int32)
for _w in range(NW):
    _IDX_PAD[_w, :SPW] = _SRC[SPW * _w:SPW * (_w + 1)]

_PERS = np.zeros((B, KP), dtype=np.float32)
_PERS[:, 0:16] = _SCALE[:, None]
_PERS[:, 16:32] = _UTY[:, None]
_PERS[:, 32:48] = _UTX[:, None]


_mesh = plsc.VectorSubcoreMesh(core_axis_name="c", subcore_axis_name="s")


@functools.partial(
    pl.kernel,
    mesh=_mesh,
    out_type=(
        jax.ShapeDtypeStruct((B, KP), jnp.float32),
        jax.ShapeDtypeStruct((B, KP), jnp.float32),
    ),
    scratch_types=[
        pltpu.VMEM((8,), jnp.int32),
        pltpu.VMEM((8, ROW), jnp.float32),
        pltpu.VMEM((KP,), jnp.float32),
        pltpu.VMEM((KP,), jnp.float32),
        pltpu.VMEM((KP,), jnp.float32),
        pltpu.SemaphoreType.DMA,
    ],
    compiler_params=pltpu.CompilerParams(needs_layout_passes=False),
)
def _sampler_kernel(data_hbm, idx_hbm, pers_hbm, pv_hbm, tr_hbm,
                    idx_v, rows_v, pers_v, pv_row, tr_row, sem):
    wid = lax.axis_index("s") * NC + lax.axis_index("c")
    pltpu.sync_copy(idx_hbm.at[wid], idx_v)
    pltpu.async_copy(data_hbm.at[idx_v], rows_v, sem).wait()
    for t in range(SPW):
        i = SPW * wid + t
        pltpu.sync_copy(pers_hbm.at[i], pers_v)
        scale_v = pers_v[pl.ds(0, 16)]
        mxh = jnp.zeros((16,), jnp.float32)
        mxw = jnp.zeros((16,), jnp.float32)
        for c in range(NCHUNK):
            x1 = rows_v[t, pl.ds(0 * KP + c * 16, 16)]
            y1 = rows_v[t, pl.ds(1 * KP + c * 16, 16)]
            x2 = rows_v[t, pl.ds(2 * KP + c * 16, 16)]
            y2 = rows_v[t, pl.ds(3 * KP + c * 16, 16)]
            vld = rows_v[t, pl.ds(4 * KP + c * 16, 16)]
            sw = (x2 - x1) * scale_v
            sh = (y2 - y1) * scale_v
            mxw = jnp.maximum(mxw, sw)
            mxh = jnp.maximum(mxh, sh)
            fits = (sh <= H) & (sw <= W)
            pv_row[pl.ds(c * 16, 16)] = jnp.where(
                fits & (vld > 0.5), 1.0, 0.0).astype(jnp.float32)
        m_h = plsc.cummax(mxh)
        m_w = plsc.cummax(mxw)
        max_ty = jnp.maximum(jnp.float32(H) - m_h, 0.0)
        max_tx = jnp.maximum(jnp.float32(W) - m_w, 0.0)
        tr_row[pl.ds(0, 16)] = pers_v[pl.ds(16, 16)] * max_ty
        tr_row[pl.ds(16, 16)] = pers_v[pl.ds(32, 16)] * max_tx
        pltpu.sync_copy(tr_row, tr_hbm.at[i])
        pltpu.sync_copy(pv_row, pv_hbm.at[i])


def kernel(images, boxes, instance_valid):
    del images
    boxes_p = jnp.pad(boxes, ((0, 0), (0, KP - K), (0, 0)))
    planes = jnp.transpose(boxes_p, (0, 2, 1))
    valid_p = jnp.pad(instance_valid, ((0, 0), (0, KP - K)))
    data = jnp.concatenate(
        [planes, valid_p.astype(jnp.float32)[:, None, :]],
        axis=1).reshape(B, ROW)

    pv_f, tr = _sampler_kernel(
        data, jnp.asarray(_IDX_PAD), jnp.asarray(_PERS))

    source_idx = jnp.asarray(_SRC, dtype=jnp.int32)
    translate = jnp.stack([tr[:, 15], tr[:, 31]], axis=-1)
    scale = jnp.asarray(_SCALE, dtype=jnp.float32)
    hflip = jnp.asarray(_HFLIP)
    paste_valid = pv_f[:, :K].astype(jnp.bool_)
    return (source_idx, translate, scale, hflip, paste_valid)


# speedup vs baseline: 1.2795x; 1.2795x over previous
import functools

import numpy as np
import jax
import jax.numpy as jnp
from jax import lax
from jax.experimental import pallas as pl
from jax.experimental.pallas import tpu as pltpu
from jax.experimental.pallas import tpu_sc as plsc

B = 64
K = 100
H = 512.0
W = 512.0
NC = 2
NS = 16
NW = NC * NS
SPW = B // NW
ROW = 512
NCHUNK = 7
OUTROW = 256


def _rotl(x, d):
    return ((x << np.uint32(d)) | (x >> np.uint32(32 - d))).astype(np.uint32)


def _threefry2x32(k0, k1, x0, x1):
    x0 = x0.astype(np.uint32).copy()
    x1 = x1.astype(np.uint32).copy()
    ks = [np.uint32(k0), np.uint32(k1),
          np.uint32(k0) ^ np.uint32(k1) ^ np.uint32(0x1BD11BDA)]
    rot = [[13, 15, 26, 6], [17, 29, 16, 24]]
    x0 = (x0 + ks[0]).astype(np.uint32)
    x1 = (x1 + ks[1]).astype(np.uint32)
    for i in range(5):
        for r in rot[i % 2]:
            x0 = (x0 + x1).astype(np.uint32)
            x1 = _rotl(x1, r)
            x1 = (x1 ^ x0).astype(np.uint32)
        x0 = (x0 + ks[(i + 1) % 3]).astype(np.uint32)
        x1 = (x1 + ks[(i + 2) % 3] + np.uint32(i + 1)).astype(np.uint32)
    return x0, x1


def _splitn(k, n):
    c = np.arange(n, dtype=np.uint32)
    a, b = _threefry2x32(k[0], k[1], np.zeros(n, np.uint32), c)
    return np.stack([a, b], -1)


def _bits(k, n):
    c = np.arange(n, dtype=np.uint32)
    a, b = _threefry2x32(k[0], k[1], np.zeros(n, np.uint32), c)
    return (a ^ b).astype(np.uint32)


def _uniform01(bits):
    m = (bits >> np.uint32(9)) | np.uint32(0x3F800000)
    return np.maximum(m.view(np.float32) - np.float32(1.0), np.float32(0.0))


def _rng_constants():
    base = np.array([0, 42], dtype=np.uint32)
    ks = _splitn(base, 5)
    k1, k2 = _splitn(ks[0], 2)
    hb, lb = _bits(k1, B), _bits(k2, B)
    span = np.uint32(B - 1)
    mult = np.uint32((2**32) % (B - 1))
    r = (((hb % span) * mult + (lb % span)) % span).astype(np.int32)
    src = (r + (r >= np.arange(B, dtype=np.int32)).astype(np.int32)).astype(np.int32)
    scale = _uniform01(_bits(ks[1], B)) * np.float32(1.5) + np.float32(0.5)
    u_ty = _uniform01(_bits(ks[2], B))
    u_tx = _uniform01(_bits(ks[3], B))
    hflip = _uniform01(_bits(ks[4], B)) < np.float32(0.5)
    return src, scale, u_ty, u_tx, hflip


_SRC, _SCALE, _UTY, _UTX, _HFLIP = _rng_constants()

_IDX_PAD = np.zeros((NW, 8), dtype=np.int32)
for _w in range(NW):
    _IDX_PAD[_w, :SPW] = _SRC[SPW * _w:SPW * (_w + 1)]

_PERS = np.zeros((B, 64), dtype=np.float32)
_PERS[:, 0:16] = _SCALE[:, None]
_PERS[:, 16:32] = _UTY[:, None]
_PERS[:, 32:48] = _UTX[:, None]


_mesh = plsc.VectorSubcoreMesh(core_axis_name="c", subcore_axis_name="s")


@functools.partial(
    pl.kernel,
    mesh=_mesh,
    out_type=jax.ShapeDtypeStruct((B, OUTROW), jnp.float32),
    scratch_types=[
        pltpu.VMEM((8,), jnp.int32),
        pltpu.VMEM((SPW, ROW), jnp.float32),
        pltpu.VMEM((SPW, 64), jnp.float32),
        pltpu.VMEM((SPW, OUTROW), jnp.float32),
        pltpu.SemaphoreType.DMA,
        pltpu.SemaphoreType.DMA,
        pltpu.SemaphoreType.DMA,
    ],
    compiler_params=pltpu.CompilerParams(needs_layout_passes=False),
)
def _sampler_kernel(data_hbm, idx_hbm, pers_hbm, out_hbm,
                    idx_v, rows_v, pers_v, out_v, sem_i, sem_p, sem_g):
    wid = lax.axis_index("s") * NC + lax.axis_index("c")
    cp_i = pltpu.async_copy(idx_hbm.at[wid], idx_v, sem_i)
    cp_p = pltpu.async_copy(pers_hbm.at[pl.ds(SPW * wid, SPW)], pers_v, sem_p)
    cp_i.wait()
    cp_g = pltpu.async_copy(data_hbm.at[idx_v.at[pl.ds(0, SPW)]], rows_v, sem_g)
    cp_p.wait()
    cp_g.wait()
    lane = lax.iota(jnp.int32, 16)
    for t in range(SPW):
        scale_v = pers_v[t, pl.ds(0, 16)]
        mxh = jnp.zeros((16,), jnp.float32)
        mxw = jnp.zeros((16,), jnp.float32)
        tvec = jnp.full((16,), t, jnp.int32)
        for c in range(NCHUNK):
            pos = lane * 4 + (64 * c)
            x1 = plsc.load_gather(rows_v, [tvec, pos])
            y1 = plsc.load_gather(rows_v, [tvec, pos + 1])
            x2 = plsc.load_gather(rows_v, [tvec, pos + 2])
            y2 = plsc.load_gather(rows_v, [tvec, pos + 3])
            vld = rows_v[t, pl.ds(400 + c * 16, 16)]
            sw = (x2 - x1) * scale_v
            sh = (y2 - y1) * scale_v
            if c == NCHUNK - 1:
                live = lane < (K - 16 * (NCHUNK - 1))
                sw = jnp.where(live, sw, 0.0)
                sh = jnp.where(live, sh, 0.0)
            mxw = jnp.maximum(mxw, sw)
            mxh = jnp.maximum(mxh, sh)
            fits = (sh <= H) & (sw <= W)
            out_v[t, pl.ds(c * 16, 16)] = jnp.where(
                fits & (vld > 0.5), 1.0, 0.0).astype(jnp.float32)
        m_h = plsc.cummax(mxh)
        m_w = plsc.cummax(mxw)
        max_ty = jnp.maximum(jnp.float32(H) - m_h, 0.0)
        max_tx = jnp.maximum(jnp.float32(W) - m_w, 0.0)
        out_v[t, pl.ds(128, 16)] = pers_v[t, pl.ds(16, 16)] * max_ty
        out_v[t, pl.ds(144, 16)] = pers_v[t, pl.ds(32, 16)] * max_tx
    pltpu.sync_copy(out_v, out_hbm.at[pl.ds(SPW * wid, SPW)])


def kernel(images, boxes, instance_valid):
    del images
    data = jnp.pad(
        jnp.concatenate(
            [boxes.reshape(B, 4 * K), instance_valid.astype(jnp.float32)],
            axis=1),
        ((0, 0), (0, ROW - 5 * K)))

    out = _sampler_kernel(data, jnp.asarray(_IDX_PAD), jnp.asarray(_PERS))

    source_idx = jnp.asarray(_SRC, dtype=jnp.int32)
    translate = jnp.stack([out[:, 143], out[:, 159]], axis=-1)
    scale = jnp.asarray(_SCALE, dtype=jnp.float32)
    hflip = jnp.asarray(_HFLIP)
    paste_valid = out[:, :K].astype(jnp.bool_)
    return (source_idx, translate, scale, hflip, paste_valid)


# speedup vs baseline: 1.3044x; 1.0194x over previous
import functools

import numpy as np
import jax
import jax.numpy as jnp
from jax import lax
from jax.experimental import pallas as pl
from jax.experimental.pallas import tpu as pltpu
from jax.experimental.pallas import tpu_sc as plsc

B = 64
K = 100
H = 512.0
W = 512.0
NC = 2
NS = 16
NW = NC * NS
SPW = B // NW
ROW = 512
NCHUNK = 7
OUTROW = 256
TAIL_SCALE, TAIL_UTY, TAIL_UTX, TAIL_SRC = 4, 5, 6, 7


def _rotl(x, d):
    return ((x << np.uint32(d)) | (x >> np.uint32(32 - d))).astype(np.uint32)


def _threefry2x32(k0, k1, x0, x1):
    x0 = x0.astype(np.uint32).copy()
    x1 = x1.astype(np.uint32).copy()
    ks = [np.uint32(k0), np.uint32(k1),
          np.uint32(k0) ^ np.uint32(k1) ^ np.uint32(0x1BD11BDA)]
    rot = [[13, 15, 26, 6], [17, 29, 16, 24]]
    x0 = (x0 + ks[0]).astype(np.uint32)
    x1 = (x1 + ks[1]).astype(np.uint32)
    for i in range(5):
        for r in rot[i % 2]:
            x0 = (x0 + x1).astype(np.uint32)
            x1 = _rotl(x1, r)
            x1 = (x1 ^ x0).astype(np.uint32)
        x0 = (x0 + ks[(i + 1) % 3]).astype(np.uint32)
        x1 = (x1 + ks[(i + 2) % 3] + np.uint32(i + 1)).astype(np.uint32)
    return x0, x1


def _splitn(k, n):
    c = np.arange(n, dtype=np.uint32)
    a, b = _threefry2x32(k[0], k[1], np.zeros(n, np.uint32), c)
    return np.stack([a, b], -1)


def _bits(k, n):
    c = np.arange(n, dtype=np.uint32)
    a, b = _threefry2x32(k[0], k[1], np.zeros(n, np.uint32), c)
    return (a ^ b).astype(np.uint32)


def _uniform01(bits):
    m = (bits >> np.uint32(9)) | np.uint32(0x3F800000)
    return np.maximum(m.view(np.float32) - np.float32(1.0), np.float32(0.0))


def _rng_constants():
    base = np.array([0, 42], dtype=np.uint32)
    ks = _splitn(base, 5)
    k1, k2 = _splitn(ks[0], 2)
    hb, lb = _bits(k1, B), _bits(k2, B)
    span = np.uint32(B - 1)
    mult = np.uint32((2**32) % (B - 1))
    r = (((hb % span) * mult + (lb % span)) % span).astype(np.int32)
    src = (r + (r >= np.arange(B, dtype=np.int32)).astype(np.int32)).astype(np.int32)
    scale = _uniform01(_bits(ks[1], B)) * np.float32(1.5) + np.float32(0.5)
    u_ty = _uniform01(_bits(ks[2], B))
    u_tx = _uniform01(_bits(ks[3], B))
    hflip = _uniform01(_bits(ks[4], B)) < np.float32(0.5)
    return src, scale, u_ty, u_tx, hflip


_SRC, _SCALE, _UTY, _UTX, _HFLIP = _rng_constants()

_TAIL = np.stack(
    [_SCALE, _UTY, _UTX, _SRC.astype(np.float32)], axis=1).astype(np.float32)


_mesh = plsc.VectorSubcoreMesh(core_axis_name="c", subcore_axis_name="s")


@functools.partial(
    pl.kernel,
    mesh=_mesh,
    out_type=jax.ShapeDtypeStruct((B, OUTROW), jnp.float32),
    scratch_types=[
        pltpu.VMEM((16,), jnp.int32),
        pltpu.VMEM((SPW, ROW), jnp.float32),
        pltpu.VMEM((SPW, ROW), jnp.float32),
        pltpu.VMEM((SPW, OUTROW), jnp.float32),
        pltpu.SemaphoreType.DMA,
    ],
    compiler_params=pltpu.CompilerParams(needs_layout_passes=False),
)
def _sampler_kernel(data_hbm, out_hbm,
                    idx_v, own_v, rows_v, out_v, sem):
    wid = lax.axis_index("s") * NC + lax.axis_index("c")
    lane = lax.iota(jnp.int32, 16)
    pltpu.sync_copy(data_hbm.at[pl.ds(SPW * wid, SPW)], own_v)
    tail0 = own_v[0, pl.ds(ROW - 16, 16)]
    tail1 = own_v[1, pl.ds(ROW - 16, 16)]
    i0 = tail0[TAIL_SRC].astype(jnp.int32)
    i1 = tail1[TAIL_SRC].astype(jnp.int32)
    idx_v[...] = jnp.where(lane < 1, i0, i1)
    pltpu.async_copy(
        data_hbm.at[idx_v.at[pl.ds(0, SPW)]], rows_v, sem).wait()
    for t in range(SPW):
        tail = (tail0, tail1)[t]
        scale_v = tail[TAIL_SCALE]
        mxh = jnp.zeros((16,), jnp.float32)
        mxw = jnp.zeros((16,), jnp.float32)
        tvec = jnp.full((16,), t, jnp.int32)
        for c in range(NCHUNK):
            pos = lane * 4 + (64 * c)
            x1 = plsc.load_gather(rows_v, [tvec, pos])
            y1 = plsc.load_gather(rows_v, [tvec, pos + 1])
            x2 = plsc.load_gather(rows_v, [tvec, pos + 2])
            y2 = plsc.load_gather(rows_v, [tvec, pos + 3])
            vld = rows_v[t, pl.ds(400 + c * 16, 16)]
            sw = (x2 - x1) * scale_v
            sh = (y2 - y1) * scale_v
            if c == NCHUNK - 1:
                live = lane < (K - 16 * (NCHUNK - 1))
                sw = jnp.where(live, sw, 0.0)
                sh = jnp.where(live, sh, 0.0)
            mxw = jnp.maximum(mxw, sw)
            mxh = jnp.maximum(mxh, sh)
            fits = (sh <= H) & (sw <= W)
            out_v[t, pl.ds(c * 16, 16)] = jnp.where(
                fits & (vld > 0.5), 1.0, 0.0).astype(jnp.float32)
        m_h = plsc.cummax(mxh)
        m_w = plsc.cummax(mxw)
        max_ty = jnp.maximum(jnp.float32(H) - m_h, 0.0)
        max_tx = jnp.maximum(jnp.float32(W) - m_w, 0.0)
        out_v[t, pl.ds(128, 16)] = max_ty * tail[TAIL_UTY]
        out_v[t, pl.ds(144, 16)] = max_tx * tail[TAIL_UTX]
    pltpu.sync_copy(out_v, out_hbm.at[pl.ds(SPW * wid, SPW)])


def kernel(images, boxes, instance_valid):
    del images
    data = jnp.pad(
        jnp.concatenate(
            [boxes.reshape(B, 4 * K), instance_valid.astype(jnp.float32),
             jnp.asarray(_TAIL)],
            axis=1),
        ((0, 0), (0, ROW - 5 * K - 4)))

    out = _sampler_kernel(data)

    source_idx = jnp.asarray(_SRC, dtype=jnp.int32)
    translate = jnp.stack([out[:, 143], out[:, 159]], axis=-1)
    scale = jnp.asarray(_SCALE, dtype=jnp.float32)
    hflip = jnp.asarray(_HFLIP)
    paste_valid = out[:, :K].astype(jnp.bool_)
    return (source_idx, translate, scale, hflip, paste_valid)


# speedup vs baseline: 1.3112x; 1.0053x over previous
import functools

import numpy as np
import jax
import jax.numpy as jnp
from jax import lax
from jax.experimental import pallas as pl
from jax.experimental.pallas import tpu as pltpu
from jax.experimental.pallas import tpu_sc as plsc

B = 64
K = 100
H = 512.0
W = 512.0
NC = 2
NS = 16
NW = NC * NS
SPW = B // NW
ROW = 512
NCHUNK = 7
OUTROW = 256
TAIL_SCALE, TAIL_UTY, TAIL_UTX, TAIL_SRC = 4, 5, 6, 7


def _rotl(x, d):
    return ((x << np.uint32(d)) | (x >> np.uint32(32 - d))).astype(np.uint32)


def _threefry2x32(k0, k1, x0, x1):
    x0 = x0.astype(np.uint32).copy()
    x1 = x1.astype(np.uint32).copy()
    ks = [np.uint32(k0), np.uint32(k1),
          np.uint32(k0) ^ np.uint32(k1) ^ np.uint32(0x1BD11BDA)]
    rot = [[13, 15, 26, 6], [17, 29, 16, 24]]
    x0 = (x0 + ks[0]).astype(np.uint32)
    x1 = (x1 + ks[1]).astype(np.uint32)
    for i in range(5):
        for r in rot[i % 2]:
            x0 = (x0 + x1).astype(np.uint32)
            x1 = _rotl(x1, r)
            x1 = (x1 ^ x0).astype(np.uint32)
        x0 = (x0 + ks[(i + 1) % 3]).astype(np.uint32)
        x1 = (x1 + ks[(i + 2) % 3] + np.uint32(i + 1)).astype(np.uint32)
    return x0, x1


def _splitn(k, n):
    c = np.arange(n, dtype=np.uint32)
    a, b = _threefry2x32(k[0], k[1], np.zeros(n, np.uint32), c)
    return np.stack([a, b], -1)


def _bits(k, n):
    c = np.arange(n, dtype=np.uint32)
    a, b = _threefry2x32(k[0], k[1], np.zeros(n, np.uint32), c)
    return (a ^ b).astype(np.uint32)


def _uniform01(bits):
    m = (bits >> np.uint32(9)) | np.uint32(0x3F800000)
    return np.maximum(m.view(np.float32) - np.float32(1.0), np.float32(0.0))


def _rng_constants():
    base = np.array([0, 42], dtype=np.uint32)
    ks = _splitn(base, 5)
    k1, k2 = _splitn(ks[0], 2)
    hb, lb = _bits(k1, B), _bits(k2, B)
    span = np.uint32(B - 1)
    mult = np.uint32((2**32) % (B - 1))
    r = (((hb % span) * mult + (lb % span)) % span).astype(np.int32)
    src = (r + (r >= np.arange(B, dtype=np.int32)).astype(np.int32)).astype(np.int32)
    scale = _uniform01(_bits(ks[1], B)) * np.float32(1.5) + np.float32(0.5)
    u_ty = _uniform01(_bits(ks[2], B))
    u_tx = _uniform01(_bits(ks[3], B))
    hflip = _uniform01(_bits(ks[4], B)) < np.float32(0.5)
    return src, scale, u_ty, u_tx, hflip


_SRC, _SCALE, _UTY, _UTX, _HFLIP = _rng_constants()

_TAIL = np.stack(
    [_SCALE, _UTY, _UTX, _SRC.astype(np.float32)], axis=1).astype(np.float32)


_mesh = plsc.VectorSubcoreMesh(core_axis_name="c", subcore_axis_name="s")


@functools.partial(
    pl.kernel,
    mesh=_mesh,
    out_type=jax.ShapeDtypeStruct((B, OUTROW), jnp.float32),
    scratch_types=[
        pltpu.VMEM((16,), jnp.int32),
        pltpu.VMEM((SPW, ROW), jnp.float32),
        pltpu.VMEM((SPW, ROW), jnp.float32),
        pltpu.VMEM((SPW, OUTROW), jnp.float32),
        pltpu.SemaphoreType.DMA,
    ],
    compiler_params=pltpu.CompilerParams(
        needs_layout_passes=False,
        skip_device_barrier=True,
        disable_semaphore_checks=True,
    ),
)
def _sampler_kernel(data_hbm, out_hbm,
                    idx_v, own_v, rows_v, out_v, sem):
    wid = lax.axis_index("s") * NC + lax.axis_index("c")
    lane = lax.iota(jnp.int32, 16)
    pltpu.sync_copy(data_hbm.at[pl.ds(SPW * wid, SPW)], own_v)
    tail0 = own_v[0, pl.ds(ROW - 16, 16)]
    tail1 = own_v[1, pl.ds(ROW - 16, 16)]
    i0 = tail0[TAIL_SRC].astype(jnp.int32)
    i1 = tail1[TAIL_SRC].astype(jnp.int32)
    idx_v[...] = jnp.where(lane < 1, i0, i1)
    pltpu.async_copy(
        data_hbm.at[idx_v.at[pl.ds(0, SPW)]], rows_v, sem).wait()
    for t in range(SPW):
        tail = (tail0, tail1)[t]
        scale_v = tail[TAIL_SCALE]
        mxh = jnp.zeros((16,), jnp.float32)
        mxw = jnp.zeros((16,), jnp.float32)
        tvec = jnp.full((16,), t, jnp.int32)
        for c in range(NCHUNK):
            pos = lane * 4 + (64 * c)
            x1 = plsc.load_gather(rows_v, [tvec, pos])
            y1 = plsc.load_gather(rows_v, [tvec, pos + 1])
            x2 = plsc.load_gather(rows_v, [tvec, pos + 2])
            y2 = plsc.load_gather(rows_v, [tvec, pos + 3])
            vld = rows_v[t, pl.ds(400 + c * 16, 16)]
            sw = (x2 - x1) * scale_v
            sh = (y2 - y1) * scale_v
            if c == NCHUNK - 1:
                live = lane < (K - 16 * (NCHUNK - 1))
                sw = jnp.where(live, sw, 0.0)
                sh = jnp.where(live, sh, 0.0)
            mxw = jnp.maximum(mxw, sw)
            mxh = jnp.maximum(mxh, sh)
            fits = (sh <= H) & (sw <= W)
            out_v[t, pl.ds(c * 16, 16)] = jnp.where(
                fits & (vld > 0.5), 1.0, 0.0).astype(jnp.float32)
        m_h = plsc.cummax(mxh)
        m_w = plsc.cummax(mxw)
        max_ty = jnp.maximum(jnp.float32(H) - m_h, 0.0)
        max_tx = jnp.maximum(jnp.float32(W) - m_w, 0.0)
        out_v[t, pl.ds(128, 16)] = max_ty * tail[TAIL_UTY]
        out_v[t, pl.ds(144, 16)] = max_tx * tail[TAIL_UTX]
    pltpu.sync_copy(out_v, out_hbm.at[pl.ds(SPW * wid, SPW)])


def kernel(images, boxes, instance_valid):
    del images
    data = jnp.pad(
        jnp.concatenate(
            [boxes.reshape(B, 4 * K), instance_valid.astype(jnp.float32),
             jnp.asarray(_TAIL)],
            axis=1),
        ((0, 0), (0, ROW - 5 * K - 4)))

    out = _sampler_kernel(data)

    source_idx = jnp.asarray(_SRC, dtype=jnp.int32)
    translate = jnp.stack([out[:, 143], out[:, 159]], axis=-1)
    scale = jnp.asarray(_SCALE, dtype=jnp.float32)
    hflip = jnp.asarray(_HFLIP)
    paste_valid = out[:, :K].astype(jnp.bool_)
    return (source_idx, translate, scale, hflip, paste_valid)


# speedup vs baseline: 1.3253x; 1.0108x over previous
import functools

import numpy as np
import jax
import jax.numpy as jnp
from jax import lax
from jax.experimental import pallas as pl
from jax.experimental.pallas import tpu as pltpu
from jax.experimental.pallas import tpu_sc as plsc

B = 64
K = 100
H = 512.0
W = 512.0
NC = 2
NS = 16
NW = NC * NS
SPW = B // NW
ROW = 512
NCHUNK = 7
OUTROW = 256
TAIL_SCALE, TAIL_UTY, TAIL_UTX, TAIL_SRC = 4, 5, 6, 7


def _rotl(x, d):
    return ((x << np.uint32(d)) | (x >> np.uint32(32 - d))).astype(np.uint32)


def _threefry2x32(k0, k1, x0, x1):
    x0 = x0.astype(np.uint32).copy()
    x1 = x1.astype(np.uint32).copy()
    ks = [np.uint32(k0), np.uint32(k1),
          np.uint32(k0) ^ np.uint32(k1) ^ np.uint32(0x1BD11BDA)]
    rot = [[13, 15, 26, 6], [17, 29, 16, 24]]
    x0 = (x0 + ks[0]).astype(np.uint32)
    x1 = (x1 + ks[1]).astype(np.uint32)
    for i in range(5):
        for r in rot[i % 2]:
            x0 = (x0 + x1).astype(np.uint32)
            x1 = _rotl(x1, r)
            x1 = (x1 ^ x0).astype(np.uint32)
        x0 = (x0 + ks[(i + 1) % 3]).astype(np.uint32)
        x1 = (x1 + ks[(i + 2) % 3] + np.uint32(i + 1)).astype(np.uint32)
    return x0, x1


def _splitn(k, n):
    c = np.arange(n, dtype=np.uint32)
    a, b = _threefry2x32(k[0], k[1], np.zeros(n, np.uint32), c)
    return np.stack([a, b], -1)


def _bits(k, n):
    c = np.arange(n, dtype=np.uint32)
    a, b = _threefry2x32(k[0], k[1], np.zeros(n, np.uint32), c)
    return (a ^ b).astype(np.uint32)


def _uniform01(bits):
    m = (bits >> np.uint32(9)) | np.uint32(0x3F800000)
    return np.maximum(m.view(np.float32) - np.float32(1.0), np.float32(0.0))


def _rng_constants():
    base = np.array([0, 42], dtype=np.uint32)
    ks = _splitn(base, 5)
    k1, k2 = _splitn(ks[0], 2)
    hb, lb = _bits(k1, B), _bits(k2, B)
    span = np.uint32(B - 1)
    mult = np.uint32((2**32) % (B - 1))
    r = (((hb % span) * mult + (lb % span)) % span).astype(np.int32)
    src = (r + (r >= np.arange(B, dtype=np.int32)).astype(np.int32)).astype(np.int32)
    scale = _uniform01(_bits(ks[1], B)) * np.float32(1.5) + np.float32(0.5)
    u_ty = _uniform01(_bits(ks[2], B))
    u_tx = _uniform01(_bits(ks[3], B))
    hflip = _uniform01(_bits(ks[4], B)) < np.float32(0.5)
    return src, scale, u_ty, u_tx, hflip


_SRC, _SCALE, _UTY, _UTX, _HFLIP = _rng_constants()

_TAIL = np.stack(
    [_SCALE, _UTY, _UTX, _SRC.astype(np.float32)], axis=1).astype(np.float32)


_mesh = plsc.VectorSubcoreMesh(core_axis_name="c", subcore_axis_name="s")


@functools.partial(
    pl.kernel,
    mesh=_mesh,
    out_type=jax.ShapeDtypeStruct((B, OUTROW), jnp.float32),
    scratch_types=[
        pltpu.VMEM((16,), jnp.int32),
        pltpu.VMEM((SPW, ROW), jnp.float32),
        pltpu.VMEM((SPW, ROW), jnp.float32),
        pltpu.VMEM((SPW, OUTROW), jnp.float32),
        pltpu.SemaphoreType.DMA,
    ],
    compiler_params=pltpu.CompilerParams(
        needs_layout_passes=False,
        skip_device_barrier=True,
        disable_semaphore_checks=True,
    ),
)
def _sampler_kernel(data_hbm, out_hbm,
                    idx_v, own_v, rows_v, out_v, sem):
    wid = lax.axis_index("s") * NC + lax.axis_index("c")
    lane = lax.iota(jnp.int32, 16)
    pltpu.sync_copy(data_hbm.at[pl.ds(SPW * wid, SPW)], own_v)
    tail0 = own_v[0, pl.ds(ROW - 16, 16)]
    tail1 = own_v[1, pl.ds(ROW - 16, 16)]
    i0 = tail0[TAIL_SRC].astype(jnp.int32)
    i1 = tail1[TAIL_SRC].astype(jnp.int32)
    idx_v[...] = jnp.where(lane < 1, i0, i1)
    pltpu.async_copy(
        data_hbm.at[idx_v.at[pl.ds(0, SPW)]], rows_v, sem).wait()

    def sample_body(t, carry):
        tail = own_v[t, pl.ds(ROW - 16, 16)]
        scale_s = tail[TAIL_SCALE]
        tvec = jnp.zeros((16,), jnp.int32) + t

        def chunk_body(c, mx):
            mxw, mxh = mx
            pos = lane * 4 + c * 64
            x1 = plsc.load_gather(rows_v, [tvec, pos])
            y1 = plsc.load_gather(rows_v, [tvec, pos + 1])
            x2 = plsc.load_gather(rows_v, [tvec, pos + 2])
            y2 = plsc.load_gather(rows_v, [tvec, pos + 3])
            vld = rows_v[t, pl.ds(400 + c * 16, 16)]
            sw = (x2 - x1) * scale_s
            sh = (y2 - y1) * scale_s
            live = (c * 16 + lane) < K
            fits = (sh <= H) & (sw <= W)
            out_v[t, pl.ds(c * 16, 16)] = jnp.where(
                fits & (vld > 0.5), 1.0, 0.0).astype(jnp.float32)
            return (jnp.maximum(mxw, jnp.where(live, sw, 0.0)),
                    jnp.maximum(mxh, jnp.where(live, sh, 0.0)))

        z = jnp.zeros((16,), jnp.float32)
        mxw, mxh = lax.fori_loop(0, NCHUNK, chunk_body, (z, z))
        m_h = plsc.cummax(mxh)
        m_w = plsc.cummax(mxw)
        max_ty = jnp.maximum(jnp.float32(H) - m_h, 0.0)
        max_tx = jnp.maximum(jnp.float32(W) - m_w, 0.0)
        out_v[t, pl.ds(128, 16)] = max_ty * tail[TAIL_UTY]
        out_v[t, pl.ds(144, 16)] = max_tx * tail[TAIL_UTX]
        return carry

    lax.fori_loop(0, SPW, sample_body, 0)
    pltpu.sync_copy(out_v, out_hbm.at[pl.ds(SPW * wid, SPW)])


def kernel(images, boxes, instance_valid):
    del images
    data = jnp.pad(
        jnp.concatenate(
            [boxes.reshape(B, 4 * K), instance_valid.astype(jnp.float32),
             jnp.asarray(_TAIL)],
            axis=1),
        ((0, 0), (0, ROW - 5 * K - 4)))

    out = _sampler_kernel(data)

    source_idx = jnp.asarray(_SRC, dtype=jnp.int32)
    translate = jnp.stack([out[:, 143], out[:, 159]], axis=-1)
    scale = jnp.asarray(_SCALE, dtype=jnp.float32)
    hflip = jnp.asarray(_HFLIP)
    paste_valid = out[:, :K].astype(jnp.bool_)
    return (source_idx, translate, scale, hflip, paste_valid)


# speedup vs baseline: 1.3445x; 1.0144x over previous
import functools

import numpy as np
import jax
import jax.numpy as jnp
from jax import lax
from jax.experimental import pallas as pl
from jax.experimental.pallas import tpu as pltpu
from jax.experimental.pallas import tpu_sc as plsc

B = 64
K = 100
H = 512.0
W = 512.0
NC = 2
NS = 16
NW = NC * NS
SPW = B // NW
ROW = 512
NCHUNK = 7
OUTROW = 256
TAIL_SCALE, TAIL_UTY, TAIL_UTX, TAIL_SRC, TAIL_HF = 4, 5, 6, 7, 8


def _rotl(x, d):
    return ((x << np.uint32(d)) | (x >> np.uint32(32 - d))).astype(np.uint32)


def _threefry2x32(k0, k1, x0, x1):
    x0 = x0.astype(np.uint32).copy()
    x1 = x1.astype(np.uint32).copy()
    ks = [np.uint32(k0), np.uint32(k1),
          np.uint32(k0) ^ np.uint32(k1) ^ np.uint32(0x1BD11BDA)]
    rot = [[13, 15, 26, 6], [17, 29, 16, 24]]
    x0 = (x0 + ks[0]).astype(np.uint32)
    x1 = (x1 + ks[1]).astype(np.uint32)
    for i in range(5):
        for r in rot[i % 2]:
            x0 = (x0 + x1).astype(np.uint32)
            x1 = _rotl(x1, r)
            x1 = (x1 ^ x0).astype(np.uint32)
        x0 = (x0 + ks[(i + 1) % 3]).astype(np.uint32)
        x1 = (x1 + ks[(i + 2) % 3] + np.uint32(i + 1)).astype(np.uint32)
    return x0, x1


def _splitn(k, n):
    c = np.arange(n, dtype=np.uint32)
    a, b = _threefry2x32(k[0], k[1], np.zeros(n, np.uint32), c)
    return np.stack([a, b], -1)


def _bits(k, n):
    c = np.arange(n, dtype=np.uint32)
    a, b = _threefry2x32(k[0], k[1], np.zeros(n, np.uint32), c)
    return (a ^ b).astype(np.uint32)


def _uniform01(bits):
    m = (bits >> np.uint32(9)) | np.uint32(0x3F800000)
    return np.maximum(m.view(np.float32) - np.float32(1.0), np.float32(0.0))


def _rng_constants():
    base = np.array([0, 42], dtype=np.uint32)
    ks = _splitn(base, 5)
    k1, k2 = _splitn(ks[0], 2)
    hb, lb = _bits(k1, B), _bits(k2, B)
    span = np.uint32(B - 1)
    mult = np.uint32((2**32) % (B - 1))
    r = (((hb % span) * mult + (lb % span)) % span).astype(np.int32)
    src = (r + (r >= np.arange(B, dtype=np.int32)).astype(np.int32)).astype(np.int32)
    scale = _uniform01(_bits(ks[1], B)) * np.float32(1.5) + np.float32(0.5)
    u_ty = _uniform01(_bits(ks[2], B))
    u_tx = _uniform01(_bits(ks[3], B))
    hflip = _uniform01(_bits(ks[4], B)) < np.float32(0.5)
    return src, scale, u_ty, u_tx, hflip


_SRC, _SCALE, _UTY, _UTX, _HFLIP = _rng_constants()

_TAIL = np.stack(
    [_SCALE, _UTY, _UTX, _SRC.astype(np.float32),
     _HFLIP.astype(np.float32)], axis=1).astype(np.float32)


_mesh = plsc.VectorSubcoreMesh(core_axis_name="c", subcore_axis_name="s")


@functools.partial(
    pl.kernel,
    mesh=_mesh,
    out_type=jax.ShapeDtypeStruct((B, OUTROW), jnp.float32),
    scratch_types=[
        pltpu.VMEM((16,), jnp.int32),
        pltpu.VMEM((SPW, ROW), jnp.float32),
        pltpu.VMEM((SPW, ROW), jnp.float32),
        pltpu.VMEM((SPW, OUTROW), jnp.float32),
        pltpu.SemaphoreType.DMA,
    ],
    compiler_params=pltpu.CompilerParams(
        needs_layout_passes=False,
        skip_device_barrier=True,
        disable_semaphore_checks=True,
    ),
)
def _sampler_kernel(data_hbm, out_hbm,
                    idx_v, own_v, rows_v, out_v, sem):
    wid = lax.axis_index("s") * NC + lax.axis_index("c")
    lane = lax.iota(jnp.int32, 16)
    pltpu.sync_copy(data_hbm.at[pl.ds(SPW * wid, SPW)], own_v)
    tail0 = own_v[0, pl.ds(ROW - 16, 16)]
    tail1 = own_v[1, pl.ds(ROW - 16, 16)]
    i0 = tail0[TAIL_SRC].astype(jnp.int32)
    i1 = tail1[TAIL_SRC].astype(jnp.int32)
    idx_v[...] = jnp.where(lane < 1, i0, i1)
    pltpu.async_copy(
        data_hbm.at[idx_v.at[pl.ds(0, SPW)]], rows_v, sem).wait()

    def sample_body(t, carry):
        tail = own_v[t, pl.ds(ROW - 16, 16)]
        scale_s = tail[TAIL_SCALE]
        tvec = jnp.zeros((16,), jnp.int32) + t

        def chunk_body(c, mx):
            mxw, mxh = mx
            pos = lane * 4 + c * 64
            x1 = plsc.load_gather(rows_v, [tvec, pos])
            y1 = plsc.load_gather(rows_v, [tvec, pos + 1])
            x2 = plsc.load_gather(rows_v, [tvec, pos + 2])
            y2 = plsc.load_gather(rows_v, [tvec, pos + 3])
            vld = rows_v[t, pl.ds(400 + c * 16, 16)]
            sw = (x2 - x1) * scale_s
            sh = (y2 - y1) * scale_s
            live = (c * 16 + lane) < K
            fits = (sh <= H) & (sw <= W)
            out_v[t, pl.ds(c * 16, 16)] = jnp.where(
                fits & (vld > 0.5), 1.0, 0.0).astype(jnp.float32)
            return (jnp.maximum(mxw, jnp.where(live, sw, 0.0)),
                    jnp.maximum(mxh, jnp.where(live, sh, 0.0)))

        z = jnp.zeros((16,), jnp.float32)
        mxw, mxh = lax.fori_loop(0, NCHUNK, chunk_body, (z, z))
        m_h = plsc.cummax(mxh)[15]
        m_w = plsc.cummax(mxw)[15]
        ty_s = jnp.maximum(jnp.float32(H) - m_h, 0.0) * tail[TAIL_UTY]
        tx_s = jnp.maximum(jnp.float32(W) - m_w, 0.0) * tail[TAIL_UTX]
        sval = jnp.where(lane == 0, ty_s,
               jnp.where(lane == 1, tx_s,
               jnp.where(lane == 2, tail[TAIL_SCALE],
               jnp.where(lane == 3, tail[TAIL_SRC],
               jnp.where(lane == 4, tail[TAIL_HF], 0.0)))))
        out_v[t, pl.ds(128, 16)] = sval
        return carry

    lax.fori_loop(0, SPW, sample_body, 0)
    pltpu.sync_copy(out_v, out_hbm.at[pl.ds(SPW * wid, SPW)])


def kernel(images, boxes, instance_valid):
    del images
    data = jnp.pad(
        jnp.concatenate(
            [boxes.reshape(B, 4 * K), instance_valid.astype(jnp.float32),
             jnp.asarray(_TAIL)],
            axis=1),
        ((0, 0), (0, ROW - 5 * K - 5)))

    out = _sampler_kernel(data)

    source_idx = out[:, 131].astype(jnp.int32)
    translate = out[:, 128:130]
    scale = out[:, 130]
    hflip = out[:, 132] > 0.5
    paste_valid = out[:, :K].astype(jnp.bool_)
    return (source_idx, translate, scale, hflip, paste_valid)
